# baseline (device time: 211425 ns/iter reference)
import jax
import jax.numpy as jnp
from jax import lax
from jax.experimental import pallas as pl
from jax.experimental.pallas import tpu as pltpu

N_CHUNKS = 4


def kernel(x):
    m, n = x.shape
    half = n // 2
    rows = m // N_CHUNKS

    def body(x_ref, out_ref, in_buf, send_buf, keep_buf,
             load_sems, keep_sems, send_sems, recv_sems):
        my_x = lax.axis_index("x")
        my_y = lax.axis_index("y")
        my_z = lax.axis_index("z")
        peer_x = 1 - my_x

        barrier_sem = pltpu.get_barrier_semaphore()
        pl.semaphore_signal(
            barrier_sem, inc=1,
            device_id=(peer_x, my_y, my_z),
            device_id_type=pl.DeviceIdType.MESH,
        )
        pl.semaphore_wait(barrier_sem, 1)

        def load(k):
            return pltpu.make_async_copy(
                x_ref.at[pl.ds(k * rows, rows), :],
                in_buf.at[k % 2],
                load_sems.at[k % 2],
            )

        rdmas = []
        keeps = []
        load(0).start()
        for k in range(N_CHUNKS):
            if k + 1 < N_CHUNKS:
                load(k + 1).start()
            load(k).wait()

            slot = k % 2

            @pl.when(my_x == 0)
            def _():
                send_buf[k] = in_buf[slot, :, half:]
                keep_buf[k] = in_buf[slot, :, :half]

            @pl.when(my_x == 1)
            def _():
                send_buf[k] = in_buf[slot, :, :half]
                keep_buf[k] = in_buf[slot, :, half:]

            rdma = pltpu.make_async_remote_copy(
                src_ref=send_buf.at[k],
                dst_ref=out_ref.at[pl.ds(my_x * m + k * rows, rows), :],
                send_sem=send_sems.at[k],
                recv_sem=recv_sems.at[k],
                device_id=(peer_x, my_y, my_z),
                device_id_type=pl.DeviceIdType.MESH,
            )
            rdma.start()
            rdmas.append(rdma)

            keep = pltpu.make_async_copy(
                keep_buf.at[k],
                out_ref.at[pl.ds(my_x * m + k * rows, rows), :],
                keep_sems.at[k],
            )
            keep.start()
            keeps.append(keep)

        for keep in keeps:
            keep.wait()
        for rdma in rdmas:
            rdma.wait()

    return pl.pallas_call(
        body,
        out_shape=jax.ShapeDtypeStruct((2 * m, half), x.dtype),
        in_specs=[pl.BlockSpec(memory_space=pl.ANY)],
        out_specs=pl.BlockSpec(memory_space=pl.ANY),
        scratch_shapes=[
            pltpu.VMEM((2, rows, n), x.dtype),
            pltpu.VMEM((N_CHUNKS, rows, half), x.dtype),
            pltpu.VMEM((N_CHUNKS, rows, half), x.dtype),
            pltpu.SemaphoreType.DMA((2,)),
            pltpu.SemaphoreType.DMA((N_CHUNKS,)),
            pltpu.SemaphoreType.DMA((N_CHUNKS,)),
            pltpu.SemaphoreType.DMA((N_CHUNKS,)),
        ],
        compiler_params=pltpu.CompilerParams(
            collective_id=0,
            vmem_limit_bytes=60 * 1024 * 1024,
        ),
    )(x)


# device time: 209811 ns/iter; 1.0077x vs baseline; 1.0077x over previous
import jax
import jax.numpy as jnp
from jax import lax
from jax.experimental import pallas as pl
from jax.experimental.pallas import tpu as pltpu

N_CHUNKS = 8


def kernel(x):
    m, n = x.shape
    half = n // 2
    rows = m // N_CHUNKS

    def body(x_ref, out_ref, in_buf, send_buf, keep_buf,
             load_sems, keep_sems, send_sems, recv_sems):
        my_x = lax.axis_index("x")
        my_y = lax.axis_index("y")
        my_z = lax.axis_index("z")
        peer_x = 1 - my_x

        barrier_sem = pltpu.get_barrier_semaphore()
        pl.semaphore_signal(
            barrier_sem, inc=1,
            device_id=(peer_x, my_y, my_z),
            device_id_type=pl.DeviceIdType.MESH,
        )
        pl.semaphore_wait(barrier_sem, 1)

        def load(k):
            return pltpu.make_async_copy(
                x_ref.at[pl.ds(k * rows, rows), :],
                in_buf.at[k % 2],
                load_sems.at[k % 2],
            )

        rdmas = []
        keeps = []
        load(0).start()
        for k in range(N_CHUNKS):
            if k + 1 < N_CHUNKS:
                load(k + 1).start()
            load(k).wait()

            slot = k % 2

            @pl.when(my_x == 0)
            def _():
                send_buf[k] = in_buf[slot, :, half:]
                keep_buf[k] = in_buf[slot, :, :half]

            @pl.when(my_x == 1)
            def _():
                send_buf[k] = in_buf[slot, :, :half]
                keep_buf[k] = in_buf[slot, :, half:]

            rdma = pltpu.make_async_remote_copy(
                src_ref=send_buf.at[k],
                dst_ref=out_ref.at[pl.ds(my_x * m + k * rows, rows), :],
                send_sem=send_sems.at[k],
                recv_sem=recv_sems.at[k],
                device_id=(peer_x, my_y, my_z),
                device_id_type=pl.DeviceIdType.MESH,
            )
            rdma.start()
            rdmas.append(rdma)

            keep = pltpu.make_async_copy(
                keep_buf.at[k],
                out_ref.at[pl.ds(my_x * m + k * rows, rows), :],
                keep_sems.at[k],
            )
            keep.start()
            keeps.append(keep)

        for keep in keeps:
            keep.wait()
        for rdma in rdmas:
            rdma.wait()

    return pl.pallas_call(
        body,
        out_shape=jax.ShapeDtypeStruct((2 * m, half), x.dtype),
        in_specs=[pl.BlockSpec(memory_space=pl.ANY)],
        out_specs=pl.BlockSpec(memory_space=pl.ANY),
        scratch_shapes=[
            pltpu.VMEM((2, rows, n), x.dtype),
            pltpu.VMEM((N_CHUNKS, rows, half), x.dtype),
            pltpu.VMEM((N_CHUNKS, rows, half), x.dtype),
            pltpu.SemaphoreType.DMA((2,)),
            pltpu.SemaphoreType.DMA((N_CHUNKS,)),
            pltpu.SemaphoreType.DMA((N_CHUNKS,)),
            pltpu.SemaphoreType.DMA((N_CHUNKS,)),
        ],
        compiler_params=pltpu.CompilerParams(
            collective_id=0,
            vmem_limit_bytes=60 * 1024 * 1024,
        ),
    )(x)
